# baseline (speedup 1.0000x reference)
import functools

import jax
import jax.numpy as jnp
from jax import lax
from jax.experimental import pallas as pl
from jax.experimental.pallas import tpu as pltpu
from jax.experimental.pallas import tpu_sc as plsc

_N = 1048576
_T = 1000
_NC = 2
_NS = 16
_NW = _NC * _NS
_PER_W = _N // _NW
_CHUNK = 8192
_NCHUNK = _PER_W // _CHUNK
_NOB = 3
_L = 16


def _popart_body(values_hbm, ids_hbm, w_hbm, b_hbm, s_hbm, m_hbm,
                 out_n_hbm, out_u_hbm, w_v, b_v, s_v, m_v, *rest):
    ids_bufs = list(rest[0:_NCHUNK])
    vals_bufs = list(rest[_NCHUNK:2 * _NCHUNK])
    on_bufs = list(rest[2 * _NCHUNK:2 * _NCHUNK + _NOB])
    ou_bufs = list(rest[2 * _NCHUNK + _NOB:2 * _NCHUNK + 2 * _NOB])
    sin = list(rest[2 * _NCHUNK + 2 * _NOB:3 * _NCHUNK + 2 * _NOB])
    sout = list(rest[3 * _NCHUNK + 2 * _NOB:3 * _NCHUNK + 3 * _NOB])
    tsem = rest[3 * _NCHUNK + 3 * _NOB]

    wid = lax.axis_index("s") * _NC + lax.axis_index("c")
    base = wid * _PER_W

    th = [
        pltpu.async_copy(w_hbm, w_v, tsem),
        pltpu.async_copy(b_hbm, b_v, tsem),
        pltpu.async_copy(s_hbm, s_v, tsem),
        pltpu.async_copy(m_hbm, m_v, tsem),
    ]
    in_h = []
    for ci in range(_NCHUNK):
        off = base + ci * _CHUNK
        in_h.append((
            pltpu.async_copy(ids_hbm.at[pl.ds(off, _CHUNK)], ids_bufs[ci], sin[ci]),
            pltpu.async_copy(values_hbm.at[pl.ds(off, _CHUNK)], vals_bufs[ci], sin[ci]),
        ))
    for h in th:
        h.wait()

    out_h = [None] * _NCHUNK
    for ci in range(_NCHUNK):
        bo = ci % _NOB
        in_h[ci][0].wait()
        in_h[ci][1].wait()
        if ci >= _NOB:
            out_h[ci - _NOB][0].wait()
            out_h[ci - _NOB][1].wait()

        iv, vv = ids_bufs[ci], vals_bufs[ci]
        onv, ouv = on_bufs[bo], ou_bufs[bo]

        @plsc.parallel_loop(0, _CHUNK // _L, unroll=8)
        def vec_body(j):
            sl = pl.ds(j * _L, _L)
            tid = iv[sl]
            xv = vv[sl]
            wv = plsc.load_gather(w_v, [tid])
            bv = plsc.load_gather(b_v, [tid])
            sv = plsc.load_gather(s_v, [tid])
            mv = plsc.load_gather(m_v, [tid])
            nv = wv * xv + bv
            onv[sl] = nv
            ouv[sl] = sv * nv + mv

        off = base + ci * _CHUNK
        out_h[ci] = (
            pltpu.async_copy(onv, out_n_hbm.at[pl.ds(off, _CHUNK)], sout[bo]),
            pltpu.async_copy(ouv, out_u_hbm.at[pl.ds(off, _CHUNK)], sout[bo]),
        )

    for ci in range(max(0, _NCHUNK - _NOB), _NCHUNK):
        out_h[ci][0].wait()
        out_h[ci][1].wait()


@jax.jit
def kernel(values, task_ids, w, b, sigma, mu):
    mesh = plsc.VectorSubcoreMesh(core_axis_name="c", subcore_axis_name="s")
    f = pl.kernel(
        _popart_body,
        mesh=mesh,
        out_type=[
            jax.ShapeDtypeStruct((_N,), jnp.float32),
            jax.ShapeDtypeStruct((_N,), jnp.float32),
        ],
        scratch_types=(
            [pltpu.VMEM((_T,), jnp.float32)] * 4
            + [pltpu.VMEM((_CHUNK,), jnp.int32)] * _NCHUNK
            + [pltpu.VMEM((_CHUNK,), jnp.float32)] * _NCHUNK
            + [pltpu.VMEM((_CHUNK,), jnp.float32)] * (2 * _NOB)
            + [pltpu.SemaphoreType.DMA] * (_NCHUNK + _NOB + 1)
        ),
        compiler_params=pltpu.CompilerParams(needs_layout_passes=False),
    )
    out_n, out_u = f(values, task_ids, w, b, sigma, mu)
    return (out_n, out_u)


# speedup vs baseline: 1.0388x; 1.0388x over previous
import functools

import jax
import jax.numpy as jnp
from jax import lax
from jax.experimental import pallas as pl
from jax.experimental.pallas import tpu as pltpu
from jax.experimental.pallas import tpu_sc as plsc

_N = 1048576
_T = 1000
_NC = 2
_NS = 16
_NW = _NC * _NS
_PER_W = _N // _NW
_CHUNK = 8192
_NCHUNK = _PER_W // _CHUNK
_NBUF = 2
_L = 16


def _popart_body(values_hbm, ids_hbm, w_hbm, b_hbm, s_hbm, m_hbm,
                 out_n_hbm, out_u_hbm, w_v, b_v, s_v, m_v, *rest):
    ids_bufs = list(rest[0:_NBUF])
    vals_bufs = list(rest[_NBUF:2 * _NBUF])
    on_bufs = list(rest[2 * _NBUF:3 * _NBUF])
    ou_bufs = list(rest[3 * _NBUF:4 * _NBUF])
    sin = list(rest[4 * _NBUF:5 * _NBUF])
    sout = list(rest[5 * _NBUF:6 * _NBUF])
    tsem = rest[6 * _NBUF]

    wid = lax.axis_index("s") * _NC + lax.axis_index("c")
    base = wid * _PER_W

    th = [
        pltpu.async_copy(w_hbm, w_v, tsem),
        pltpu.async_copy(b_hbm, b_v, tsem),
        pltpu.async_copy(s_hbm, s_v, tsem),
        pltpu.async_copy(m_hbm, m_v, tsem),
    ]

    def start_in(ci):
        bi = ci % _NBUF
        off = base + ci * _CHUNK
        h1 = pltpu.async_copy(ids_hbm.at[pl.ds(off, _CHUNK)], ids_bufs[bi], sin[bi])
        h2 = pltpu.async_copy(values_hbm.at[pl.ds(off, _CHUNK)], vals_bufs[bi], sin[bi])
        return (h1, h2)

    in_h = [None] * _NCHUNK
    out_h = [None] * _NCHUNK
    for ci in range(min(_NBUF - 1, _NCHUNK)):
        in_h[ci] = start_in(ci)
    for h in th:
        h.wait()

    for ci in range(_NCHUNK):
        bi = ci % _NBUF
        if ci + _NBUF - 1 < _NCHUNK:
            in_h[ci + _NBUF - 1] = start_in(ci + _NBUF - 1)
        in_h[ci][0].wait()
        in_h[ci][1].wait()
        if ci >= _NBUF:
            out_h[ci - _NBUF][0].wait()
            out_h[ci - _NBUF][1].wait()

        iv, vv = ids_bufs[bi], vals_bufs[bi]
        onv, ouv = on_bufs[bi], ou_bufs[bi]

        @plsc.parallel_loop(0, _CHUNK // _L, unroll=8)
        def vec_body(j):
            sl = pl.ds(j * _L, _L)
            tid = iv[sl]
            xv = vv[sl]
            wv = plsc.load_gather(w_v, [tid])
            bv = plsc.load_gather(b_v, [tid])
            sv = plsc.load_gather(s_v, [tid])
            mv = plsc.load_gather(m_v, [tid])
            nv = wv * xv + bv
            onv[sl] = nv
            ouv[sl] = sv * nv + mv

        off = base + ci * _CHUNK
        out_h[ci] = (
            pltpu.async_copy(onv, out_n_hbm.at[pl.ds(off, _CHUNK)], sout[bi]),
            pltpu.async_copy(ouv, out_u_hbm.at[pl.ds(off, _CHUNK)], sout[bi]),
        )

    for ci in range(max(0, _NCHUNK - _NBUF), _NCHUNK):
        out_h[ci][0].wait()
        out_h[ci][1].wait()


@jax.jit
def kernel(values, task_ids, w, b, sigma, mu):
    mesh = plsc.VectorSubcoreMesh(core_axis_name="c", subcore_axis_name="s")
    f = pl.kernel(
        _popart_body,
        mesh=mesh,
        out_type=[
            jax.ShapeDtypeStruct((_N,), jnp.float32),
            jax.ShapeDtypeStruct((_N,), jnp.float32),
        ],
        scratch_types=(
            [pltpu.VMEM((_T,), jnp.float32)] * 4
            + [pltpu.VMEM((_CHUNK,), jnp.int32)] * _NBUF
            + [pltpu.VMEM((_CHUNK,), jnp.float32)] * (3 * _NBUF)
            + [pltpu.SemaphoreType.DMA] * (2 * _NBUF + 1)
        ),
        compiler_params=pltpu.CompilerParams(needs_layout_passes=False),
    )
    out_n, out_u = f(values, task_ids, w, b, sigma, mu)
    return (out_n, out_u)


# speedup vs baseline: 1.0398x; 1.0010x over previous
import functools

import jax
import jax.numpy as jnp
from jax import lax
from jax.experimental import pallas as pl
from jax.experimental.pallas import tpu as pltpu
from jax.experimental.pallas import tpu_sc as plsc

_N = 1048576
_T = 1000
_NC = 2
_NS = 16
_NW = _NC * _NS
_PER_W = _N // _NW
_CHUNK = 8192
_NCHUNK = _PER_W // _CHUNK
_NBUF = 2
_L = 16


def _popart_body(values_hbm, ids_hbm, w_hbm, b_hbm, s_hbm, m_hbm,
                 out_n_hbm, out_u_hbm, w_v, b_v, s_v, m_v, *rest):
    ids_bufs = list(rest[0:_NBUF])
    vals_bufs = list(rest[_NBUF:2 * _NBUF])
    on_bufs = list(rest[2 * _NBUF:3 * _NBUF])
    ou_bufs = list(rest[3 * _NBUF:4 * _NBUF])
    sin = list(rest[4 * _NBUF:5 * _NBUF])
    sout = list(rest[5 * _NBUF:6 * _NBUF])
    tsem = rest[6 * _NBUF]

    wid = lax.axis_index("s") * _NC + lax.axis_index("c")
    base = wid * _PER_W

    th = [
        pltpu.async_copy(w_hbm, w_v, tsem),
        pltpu.async_copy(b_hbm, b_v, tsem),
        pltpu.async_copy(s_hbm, s_v, tsem),
        pltpu.async_copy(m_hbm, m_v, tsem),
    ]

    def start_in(ci):
        bi = ci % _NBUF
        off = base + ci * _CHUNK
        h1 = pltpu.async_copy(ids_hbm.at[pl.ds(off, _CHUNK)], ids_bufs[bi], sin[bi])
        h2 = pltpu.async_copy(values_hbm.at[pl.ds(off, _CHUNK)], vals_bufs[bi], sin[bi])
        return (h1, h2)

    in_h = [None] * _NCHUNK
    out_h = [None] * _NCHUNK
    for ci in range(min(_NBUF - 1, _NCHUNK)):
        in_h[ci] = start_in(ci)
    for h in th:
        h.wait()

    for ci in range(_NCHUNK):
        bi = ci % _NBUF
        if ci + _NBUF - 1 < _NCHUNK:
            in_h[ci + _NBUF - 1] = start_in(ci + _NBUF - 1)
        in_h[ci][0].wait()
        in_h[ci][1].wait()
        if ci >= _NBUF:
            out_h[ci - _NBUF][0].wait()
            out_h[ci - _NBUF][1].wait()

        iv, vv = ids_bufs[bi], vals_bufs[bi]
        onv, ouv = on_bufs[bi], ou_bufs[bi]

        @plsc.parallel_loop(0, _CHUNK // _L, unroll=4)
        def vec_body(j):
            sl = pl.ds(j * _L, _L)
            tid = iv[sl]
            xv = vv[sl]
            wv = plsc.load_gather(w_v, [tid])
            bv = plsc.load_gather(b_v, [tid])
            sv = plsc.load_gather(s_v, [tid])
            mv = plsc.load_gather(m_v, [tid])
            nv = wv * xv + bv
            onv[sl] = nv
            ouv[sl] = sv * nv + mv

        off = base + ci * _CHUNK
        out_h[ci] = (
            pltpu.async_copy(onv, out_n_hbm.at[pl.ds(off, _CHUNK)], sout[bi]),
            pltpu.async_copy(ouv, out_u_hbm.at[pl.ds(off, _CHUNK)], sout[bi]),
        )

    for ci in range(max(0, _NCHUNK - _NBUF), _NCHUNK):
        out_h[ci][0].wait()
        out_h[ci][1].wait()


@jax.jit
def kernel(values, task_ids, w, b, sigma, mu):
    mesh = plsc.VectorSubcoreMesh(core_axis_name="c", subcore_axis_name="s")
    f = pl.kernel(
        _popart_body,
        mesh=mesh,
        out_type=[
            jax.ShapeDtypeStruct((_N,), jnp.float32),
            jax.ShapeDtypeStruct((_N,), jnp.float32),
        ],
        scratch_types=(
            [pltpu.VMEM((_T,), jnp.float32)] * 4
            + [pltpu.VMEM((_CHUNK,), jnp.int32)] * _NBUF
            + [pltpu.VMEM((_CHUNK,), jnp.float32)] * (3 * _NBUF)
            + [pltpu.SemaphoreType.DMA] * (2 * _NBUF + 1)
        ),
        compiler_params=pltpu.CompilerParams(needs_layout_passes=False),
    )
    out_n, out_u = f(values, task_ids, w, b, sigma, mu)
    return (out_n, out_u)
